# Initial kernel scaffold; baseline (speedup 1.0000x reference)
import jax
import jax.numpy as jnp
from jax.experimental import pallas as pl


def kernel(x, adj, edges, params):
    raise NotImplementedError("write your pallas kernel here")


# speedup vs baseline: 1.0670x; 1.0670x over previous
import functools
import math

import jax
import jax.numpy as jnp
from jax.experimental import pallas as pl
from jax.experimental.pallas import tpu as pltpu

N = 10000
E = 320000
ATT = 16
ALPHA = 0.2

_BLK = 1000


def _mm_heads_body(h_ref, w_ref, b_ref, law_ref, bla_ref, raw_ref, bra_ref,
                   nh_ref, la_ref, ra_ref):
    nh = jnp.dot(h_ref[...], w_ref[...], preferred_element_type=jnp.float32)
    nh = nh + b_ref[...]
    nh_ref[...] = nh
    la_ref[...] = jnp.dot(nh, law_ref[...], preferred_element_type=jnp.float32) + bla_ref[0, 0]
    ra_ref[...] = jnp.dot(nh, raw_ref[...], preferred_element_type=jnp.float32) + bra_ref[0, 0]


def _mm_heads(h, W, B, law, bla, raw, bra):
    fin, fout = W.shape
    grid = (N // _BLK,)
    return pl.pallas_call(
        _mm_heads_body,
        grid=grid,
        in_specs=[
            pl.BlockSpec((_BLK, fin), lambda i: (i, 0)),
            pl.BlockSpec((fin, fout), lambda i: (0, 0)),
            pl.BlockSpec((1, fout), lambda i: (0, 0)),
            pl.BlockSpec((fout, 1), lambda i: (0, 0)),
            pl.BlockSpec((1, 1), lambda i: (0, 0)),
            pl.BlockSpec((fout, 1), lambda i: (0, 0)),
            pl.BlockSpec((1, 1), lambda i: (0, 0)),
        ],
        out_specs=[
            pl.BlockSpec((_BLK, fout), lambda i: (i, 0)),
            pl.BlockSpec((_BLK, 1), lambda i: (i, 0)),
            pl.BlockSpec((_BLK, 1), lambda i: (i, 0)),
        ],
        out_shape=[
            jax.ShapeDtypeStruct((N, fout), jnp.float32),
            jax.ShapeDtypeStruct((N, 1), jnp.float32),
            jax.ShapeDtypeStruct((N, 1), jnp.float32),
        ],
    )(h, W, B, law, bla.reshape(1, 1), raw, bra.reshape(1, 1))


def _gps_layer(h, aggr, src, dst, p, need_norm, thickness, fout):
    new_h, la2, ra2 = _mm_heads(h, p['W'], p['B'], p['la'], p['Bla'], p['ra'], p['Bra'])
    la = la2.reshape(-1)
    ra = ra2.reshape(-1)
    a_edge = (la[src] + ra[dst]) / math.sqrt(float(fout))
    a_edge = jnp.exp(-jax.nn.leaky_relu(a_edge, ALPHA))
    num = jax.ops.segment_sum(a_edge[:, None] * new_h[dst], src, num_segments=N)
    den = jax.ops.segment_sum(a_edge, src, num_segments=N)[:, None]
    final_h = aggr * (num / den) + (1.0 - aggr) * new_h
    if need_norm:
        final_h = (final_h - p['bn_m']) / jnp.sqrt(p['bn_v'] + 1e-5) * p['bn_g'] + p['bn_b']
    if thickness != 3:
        final_h = jax.nn.relu(final_h)
        mini = final_h @ p['W2'] + p['B2']
        h_src = mini[src]
        h_dst = mini[dst]
        fc = jnp.concatenate([h_src, h_dst, jnp.abs(h_dst - h_src)], axis=1)
        fc = jnp.tanh(fc @ p['lin1_w'].T + p['lin1_b'])
        fc = jax.nn.sigmoid(fc @ p['lin2_w'].T + p['lin2_b']).reshape(-1)
        deg = jnp.maximum(jax.ops.segment_sum(jnp.ones((E,), jnp.float32), src, num_segments=N), 1.0)
        f1 = jax.ops.segment_sum(fc, src, num_segments=N) / deg
        f2 = jax.ops.segment_sum(fc * f1[dst], src, num_segments=N) / deg
        return final_h, f2[:, None]
    return final_h, None


def kernel(x, adj, edges, params):
    src = edges[0]
    dst = edges[1]
    h = x
    aggr = jnp.ones((N, 1), jnp.float32)
    cfgs = [(True, 1, 128), (True, 2, 128), (False, 3, 64)]
    for p, (nrm, th, fo) in zip(params, cfgs):
        h, aggr = _gps_layer(h, aggr, src, dst, p, nrm, th, fo)
    return jax.nn.log_softmax(h, axis=1)


# speedup vs baseline: 3.1617x; 2.9633x over previous
import functools
import math

import jax
import jax.numpy as jnp
from jax import lax
from jax.experimental import pallas as pl
from jax.experimental.pallas import tpu as pltpu
from jax.experimental.pallas import tpu_sc as plsc

N = 10000
E = 320000
ATT = 16
ALPHA = 0.2

_BLK = 1000

_NW = 32
_EP = E // _NW
_C = 400
_NCH = _EP // _C
_NP = 10240
_SW = 640


def _build_sc2(fout):
    half = fout // 2
    nq = half // 16
    cc = E // 16
    ch = 800
    nch = cc // ch
    mesh = plsc.VectorSubcoreMesh(core_axis_name="c", subcore_axis_name="s")

    @functools.partial(
        pl.kernel,
        out_type=[
            jax.ShapeDtypeStruct((2, _NP, half), jnp.float32),
            jax.ShapeDtypeStruct((_NP,), jnp.float32),
            jax.ShapeDtypeStruct((_NP,), jnp.float32),
        ],
        mesh=mesh,
        compiler_params=pltpu.CompilerParams(needs_layout_passes=False, use_tc_tiling_on_sc=False),
        scratch_types=[
            pltpu.VMEM((N,), jnp.float32),
            pltpu.VMEM((N,), jnp.float32),
            pltpu.VMEM((ch,), jnp.int32),
            pltpu.VMEM((ch,), jnp.int32),
            pltpu.VMEM((ch,), jnp.float32),
            pltpu.VMEM((ch,), jnp.float32),
            pltpu.VMEM((ch, half), jnp.float32),
            pltpu.VMEM((128, half), jnp.float32),
            pltpu.VMEM((_SW,), jnp.float32),
            pltpu.VMEM_SHARED((_NP, half), jnp.float32),
            pltpu.VMEM_SHARED((_NP,), jnp.float32),
            pltpu.VMEM_SHARED((_NP,), jnp.float32),
            pltpu.SemaphoreType.DMA,
        ],
    )
    def sc2(nh2, la, ra, src, dst, num_p, den_p, deg_p,
            la_v, ra_v, srcb, dstb, ab, onesb, rows, zrow, zvec,
            num_sh, den_sh, deg_sh, sem):
        cid = lax.axis_index("c")
        sid = lax.axis_index("s")

        def _zrow(i, _):
            for q in range(nq):
                zrow[i, pl.ds(q * 16, 16)] = jnp.zeros((16,), jnp.float32)
            return _
        lax.fori_loop(0, 128, _zrow, None)

        def _zvec(i, _):
            zvec[pl.ds(i * 16, 16)] = jnp.zeros((16,), jnp.float32)
            return _
        lax.fori_loop(0, _SW // 16, _zvec, None)

        def _ones(i, _):
            onesb[pl.ds(i * 16, 16)] = jnp.ones((16,), jnp.float32)
            return _
        lax.fori_loop(0, ch // 16, _ones, None)

        for t in range(5):
            pltpu.sync_copy(zrow, num_sh.at[pl.ds(sid * _SW + t * 128, 128), :])

        @pl.when(cid == 0)
        def _():
            pltpu.sync_copy(zvec, den_sh.at[pl.ds(sid * _SW, _SW)])
            pltpu.sync_copy(zvec, deg_sh.at[pl.ds(sid * _SW, _SW)])

        pltpu.sync_copy(la, la_v)
        pltpu.sync_copy(ra, ra_v)
        plsc.subcore_barrier()

        def _chunk(k, _):
            base = sid * cc + k * ch
            pltpu.sync_copy(src.at[pl.ds(base, ch)], srcb)
            pltpu.sync_copy(dst.at[pl.ds(base, ch)], dstb)
            pltpu.async_copy(nh2.at[cid].at[dstb], rows, sem).wait()

            def _att(g, _):
                i16s = srcb[pl.ds(g * 16, 16)]
                i16d = dstb[pl.ds(g * 16, 16)]
                s = plsc.load_gather(la_v, [i16s]) + plsc.load_gather(ra_v, [i16d])
                ab[pl.ds(g * 16, 16)] = jnp.exp(jnp.where(s > 0, -s, -ALPHA * s))
                return _
            lax.fori_loop(0, ch // 16, _att, None)

            def _scale(e, _):
                sp = plsc.load_gather(ab, [jnp.full((16,), e, jnp.int32)])
                for q in range(nq):
                    rows[e, pl.ds(q * 16, 16)] = rows[e, pl.ds(q * 16, 16)] * sp
                return _
            lax.fori_loop(0, ch, _scale, None)

            pltpu.sync_copy(rows, num_sh.at[srcb], add=True)

            @pl.when(cid == 0)
            def _():
                pltpu.sync_copy(ab, den_sh.at[srcb], add=True)
                pltpu.sync_copy(onesb, deg_sh.at[srcb], add=True)
            return _
        lax.fori_loop(0, nch, _chunk, None)

        plsc.subcore_barrier()
        r0 = sid * _SW
        pltpu.sync_copy(num_sh.at[pl.ds(r0, _SW), :], num_p.at[cid, pl.ds(r0, _SW), :])

        @pl.when(cid == 0)
        def _():
            pltpu.sync_copy(den_sh.at[pl.ds(r0, _SW)], den_p.at[pl.ds(r0, _SW)])
            pltpu.sync_copy(deg_sh.at[pl.ds(r0, _SW)], deg_p.at[pl.ds(r0, _SW)])

    return sc2


_sc2_128 = _build_sc2(128)
_sc2_64 = _build_sc2(64)


def _mm_heads_body(h_ref, w_ref, b_ref, law_ref, bla_ref, raw_ref, bra_ref,
                   nh_ref, la_ref, ra_ref):
    nh = jnp.dot(h_ref[...], w_ref[...], preferred_element_type=jnp.float32)
    nh = nh + b_ref[...]
    nh_ref[...] = nh
    la_ref[...] = jnp.dot(nh, law_ref[...], preferred_element_type=jnp.float32) + bla_ref[0, 0]
    ra_ref[...] = jnp.dot(nh, raw_ref[...], preferred_element_type=jnp.float32) + bra_ref[0, 0]


def _mm_heads(h, W, B, law, bla, raw, bra):
    fin, fout = W.shape
    grid = (N // _BLK,)
    return pl.pallas_call(
        _mm_heads_body,
        grid=grid,
        in_specs=[
            pl.BlockSpec((_BLK, fin), lambda i: (i, 0)),
            pl.BlockSpec((fin, fout), lambda i: (0, 0)),
            pl.BlockSpec((1, fout), lambda i: (0, 0)),
            pl.BlockSpec((fout, 1), lambda i: (0, 0)),
            pl.BlockSpec((1, 1), lambda i: (0, 0)),
            pl.BlockSpec((fout, 1), lambda i: (0, 0)),
            pl.BlockSpec((1, 1), lambda i: (0, 0)),
        ],
        out_specs=[
            pl.BlockSpec((_BLK, fout), lambda i: (i, 0)),
            pl.BlockSpec((_BLK, 1), lambda i: (i, 0)),
            pl.BlockSpec((_BLK, 1), lambda i: (i, 0)),
        ],
        out_shape=[
            jax.ShapeDtypeStruct((N, fout), jnp.float32),
            jax.ShapeDtypeStruct((N, 1), jnp.float32),
            jax.ShapeDtypeStruct((N, 1), jnp.float32),
        ],
    )(h, W, B, law, bla.reshape(1, 1), raw, bra.reshape(1, 1))


def _gps_layer(h, aggr, src, dst, p, need_norm, thickness, fout):
    new_h, la2, ra2 = _mm_heads(h, p['W'], p['B'], p['la'], p['Bla'], p['ra'], p['Bra'])
    inv = 1.0 / math.sqrt(float(fout))
    la = la2.reshape(-1) * inv
    ra = ra2.reshape(-1) * inv
    sc2 = _sc2_128 if fout == 128 else _sc2_64
    half = fout // 2
    nh2 = jnp.stack([new_h[:, :half], new_h[:, half:]])
    num_p, den_p, deg_p = sc2(nh2, la, ra, src, dst)
    num = jnp.concatenate([num_p[0, :N], num_p[1, :N]], axis=1)
    den = den_p[:N, None]
    final_h = aggr * (num / den) + (1.0 - aggr) * new_h
    if need_norm:
        final_h = (final_h - p['bn_m']) / jnp.sqrt(p['bn_v'] + 1e-5) * p['bn_g'] + p['bn_b']
    if thickness != 3:
        final_h = jax.nn.relu(final_h)
        mini = final_h @ p['W2'] + p['B2']
        h_src = mini[src]
        h_dst = mini[dst]
        fc = jnp.concatenate([h_src, h_dst, jnp.abs(h_dst - h_src)], axis=1)
        fc = jnp.tanh(fc @ p['lin1_w'].T + p['lin1_b'])
        fc = jax.nn.sigmoid(fc @ p['lin2_w'].T + p['lin2_b']).reshape(-1)
        deg = jnp.maximum(deg_p[:N], 1.0)
        f1 = jax.ops.segment_sum(fc, src, num_segments=N) / deg
        f2 = jax.ops.segment_sum(fc * f1[dst], src, num_segments=N) / deg
        return final_h, f2[:, None]
    return final_h, None


def kernel(x, adj, edges, params):
    src = edges[0]
    dst = edges[1]
    h = x
    aggr = jnp.ones((N, 1), jnp.float32)
    cfgs = [(True, 1, 128), (True, 2, 128), (False, 3, 64)]
    for p, (nrm, th, fo) in zip(params, cfgs):
        h, aggr = _gps_layer(h, aggr, src, dst, p, nrm, th, fo)
    return jax.nn.log_softmax(h, axis=1)
